# baseline (speedup 1.0000x reference)
import jax
import jax.numpy as jnp
from jax.experimental import pallas as pl
from jax.experimental.pallas import tpu as pltpu

_B, _N, _D, _H = 4, 2048, 1024, 256
_S = 2 * _H
_ITERS = 3
_K1 = 64
_K2 = _S // 3
_SCALE = _H ** -0.5
_MINT = -2147483648
_EPS_LN = 1e-5


def _sortable(x):
    i = jax.lax.bitcast_convert_type(x, jnp.int32)
    return i ^ ((i >> 31) & jnp.int32(2147483647))


def _kth_largest(key, k):
    rows = key.shape[0]
    prefix = jnp.zeros((rows, 1), jnp.int32)
    for bit in range(31, -1, -1):
        c = 1 << bit
        if c >= 1 << 31:
            c -= 1 << 32
        cand = prefix | jnp.int32(c)
        thresh = cand ^ jnp.int32(_MINT)
        cnt = jnp.sum((key >= thresh).astype(jnp.int32), axis=1,
                      keepdims=True)
        prefix = jnp.where(cnt >= k, cand, prefix)
    return prefix ^ jnp.int32(_MINT)


def _topk_softmax(dots, k, inv_t):
    key = _sortable(dots)
    th = _kth_largest(key, k)
    mask = key >= th
    m = jnp.max(dots, axis=1, keepdims=True)
    e = jnp.where(mask, jnp.exp((dots - m) * inv_t), 0.0)
    return e / jnp.sum(e, axis=1, keepdims=True)


def _proj_body(x_ref, w_ref, b_ref, out_ref):
    out_ref[0] = (
        jnp.dot(x_ref[0], w_ref[...], preferred_element_type=jnp.float32)
        + b_ref[...]
    )


def _proj(x, w3, b3, bn):
    return pl.pallas_call(
        _proj_body,
        grid=(_B, _N // bn),
        in_specs=[
            pl.BlockSpec((1, bn, _D), lambda b, n: (b, n, 0)),
            pl.BlockSpec((_D, 3 * _H), lambda b, n: (0, 0)),
            pl.BlockSpec((1, 3 * _H), lambda b, n: (0, 0)),
        ],
        out_specs=pl.BlockSpec((1, bn, 3 * _H), lambda b, n: (b, n, 0)),
        out_shape=jax.ShapeDtypeStruct((_B, _N, 3 * _H), jnp.float32),
    )(x, w3, b3)


def _slots_body(inv_t_ref, k_ref, v_ref, out_ref):
    kmat = k_ref[0]
    vmat = v_ref[0]
    inv_t = inv_t_ref[0, 0]
    r = jax.lax.broadcasted_iota(jnp.int32, (_S, _H), 0)
    c = jax.lax.broadcasted_iota(jnp.int32, (_S, _H), 1)
    q = jnp.where(r == c, 1.0, 0.0) + jnp.where(r - _H == c, -1.0, 0.0)
    for _ in range(_ITERS):
        dots = jax.lax.dot_general(
            q, kmat, (((1,), (1,)), ((), ())),
            preferred_element_type=jnp.float32) * _SCALE
        p = _topk_softmax(dots, _K1, inv_t)
        s = jnp.dot(p, vmat, preferred_element_type=jnp.float32)
        nrm = jnp.sqrt(jnp.sum(s * s, axis=1, keepdims=True))
        q = s / jnp.maximum(nrm, 1e-12)
    out_ref[0] = q


def _slots(kk, vv, inv_t1):
    return pl.pallas_call(
        _slots_body,
        grid=(_B,),
        in_specs=[
            pl.BlockSpec(memory_space=pltpu.SMEM),
            pl.BlockSpec((1, _N, _H), lambda b: (b, 0, 0)),
            pl.BlockSpec((1, _N, _H), lambda b: (b, 0, 0)),
        ],
        out_specs=pl.BlockSpec((1, _S, _H), lambda b: (b, 0, 0)),
        out_shape=jax.ShapeDtypeStruct((_B, _S, _H), jnp.float32),
    )(inv_t1, kk, vv)


def _stage2_body(inv_t_ref, q_ref, s_ref, wbv_ref, bbv_ref, x_ref,
                 g_ref, beta_ref, out_ref):
    slots = s_ref[0]
    inv_t = inv_t_ref[0, 0]
    slots_v = (jnp.dot(slots.astype(jnp.bfloat16), wbv_ref[...],
                       preferred_element_type=jnp.float32) + bbv_ref[...])
    logits = jax.lax.dot_general(
        q_ref[0], slots, (((1,), (1,)), ((), ())),
        preferred_element_type=jnp.float32) * _SCALE
    p = _topk_softmax(logits, _K2, inv_t)
    y = jnp.dot(p.astype(jnp.bfloat16), slots_v.astype(jnp.bfloat16),
                preferred_element_type=jnp.float32) + x_ref[0]
    mu = jnp.mean(y, axis=1, keepdims=True)
    var = jnp.mean((y - mu) ** 2, axis=1, keepdims=True)
    out_ref[0] = ((y - mu) / jnp.sqrt(var + _EPS_LN) * g_ref[...]
                  + beta_ref[...])


def _stage2(emb_q, slots, wbv, bbv, x, g1, beta1, inv_t2):
    return pl.pallas_call(
        _stage2_body,
        grid=(_B,),
        in_specs=[
            pl.BlockSpec(memory_space=pltpu.SMEM),
            pl.BlockSpec((1, _N, _H), lambda b: (b, 0, 0)),
            pl.BlockSpec((1, _S, _H), lambda b: (b, 0, 0)),
            pl.BlockSpec((_H, _D), lambda b: (0, 0)),
            pl.BlockSpec((1, _D), lambda b: (0, 0)),
            pl.BlockSpec((1, _N, _D), lambda b: (b, 0, 0)),
            pl.BlockSpec((1, _D), lambda b: (0, 0)),
            pl.BlockSpec((1, _D), lambda b: (0, 0)),
        ],
        out_specs=pl.BlockSpec((1, _N, _D), lambda b: (b, 0, 0)),
        out_shape=jax.ShapeDtypeStruct((_B, _N, _D), jnp.float32),
    )(inv_t2, emb_q, slots, wbv, bbv, x, g1, beta1)


def _ffn_body(x_ref, w1_ref, b1_ref, w2_ref, b2_ref, g_ref, beta_ref,
              out_ref):
    x = x_ref[0]
    h = jnp.dot(x.astype(jnp.bfloat16), w1_ref[...],
                preferred_element_type=jnp.float32) + b1_ref[...]
    h = 0.5 * h * (1.0 + jax.lax.erf(h * (2.0 ** -0.5)))
    y = x + jnp.dot(h.astype(jnp.bfloat16), w2_ref[...],
                    preferred_element_type=jnp.float32) + b2_ref[...]
    mu = jnp.mean(y, axis=1, keepdims=True)
    var = jnp.mean((y - mu) ** 2, axis=1, keepdims=True)
    out_ref[0] = ((y - mu) / jnp.sqrt(var + _EPS_LN) * g_ref[...]
                  + beta_ref[...])


def _ffn(x, w1, b1, w2, b2, g2, beta2, bn):
    return pl.pallas_call(
        _ffn_body,
        grid=(_B, _N // bn),
        in_specs=[
            pl.BlockSpec((1, bn, _D), lambda b, n: (b, n, 0)),
            pl.BlockSpec((_D, 2 * _D), lambda b, n: (0, 0)),
            pl.BlockSpec((1, 2 * _D), lambda b, n: (0, 0)),
            pl.BlockSpec((2 * _D, _D), lambda b, n: (0, 0)),
            pl.BlockSpec((1, _D), lambda b, n: (0, 0)),
            pl.BlockSpec((1, _D), lambda b, n: (0, 0)),
            pl.BlockSpec((1, _D), lambda b, n: (0, 0)),
        ],
        out_specs=pl.BlockSpec((1, bn, _D), lambda b, n: (b, n, 0)),
        out_shape=jax.ShapeDtypeStruct((_B, _N, _D), jnp.float32),
    )(x, w1, b1, w2, b2, g2, beta2)


def kernel(inputs, Wk, bk, Wv, bv, WQ, bQ, Wbv, bbv, g1, beta1, W1, b1,
           W2, b2, g2, beta2, t1, t2):
    w3 = jnp.concatenate([Wk, Wv, WQ], axis=1)
    b3 = jnp.concatenate([bk, bv, bQ])[None, :]
    kvq = _proj(inputs, w3, b3, bn=512)
    kk = kvq[:, :, :_H]
    vv = kvq[:, :, _H:2 * _H]
    emb_q = kvq[:, :, 2 * _H:]
    inv_t1 = jnp.reshape(1.0 / t1, (1, 1))
    inv_t2 = jnp.reshape(1.0 / t2, (1, 1))
    slots = _slots(kk, vv, inv_t1)
    emb = _stage2(emb_q, slots, Wbv.astype(jnp.bfloat16), bbv[None, :],
                  inputs, g1[None, :], beta1[None, :], inv_t2)
    out = _ffn(emb, W1.astype(jnp.bfloat16), b1[None, :],
               W2.astype(jnp.bfloat16), b2[None, :], g2[None, :],
               beta2[None, :], bn=512)
    return out


# speedup vs baseline: 1.0006x; 1.0006x over previous
import jax
import jax.numpy as jnp
from jax.experimental import pallas as pl
from jax.experimental.pallas import tpu as pltpu

_B, _N, _D, _H = 4, 2048, 1024, 256
_S = 2 * _H
_ITERS = 3
_K1 = 64
_K2 = _S // 3
_SCALE = _H ** -0.5
_MINT = -2147483648
_EPS_LN = 1e-5


def _sortable(x):
    i = jax.lax.bitcast_convert_type(x, jnp.int32)
    return i ^ ((i >> 31) & jnp.int32(2147483647))


def _kth_largest(key, k):
    rows = key.shape[0]
    prefix = jnp.zeros((rows, 1), jnp.int32)
    for bit in range(31, -1, -1):
        c = 1 << bit
        if c >= 1 << 31:
            c -= 1 << 32
        cand = prefix | jnp.int32(c)
        thresh = cand ^ jnp.int32(_MINT)
        cnt = jnp.sum((key >= thresh).astype(jnp.int32), axis=1,
                      keepdims=True)
        prefix = jnp.where(cnt >= k, cand, prefix)
    return prefix ^ jnp.int32(_MINT)


def _topk_softmax(dots, k, inv_t):
    key = _sortable(dots)
    th = _kth_largest(key, k)
    mask = key >= th
    m = jnp.max(dots, axis=1, keepdims=True)
    e = jnp.where(mask, jnp.exp((dots - m) * inv_t), 0.0)
    return e / jnp.sum(e, axis=1, keepdims=True)


def _proj_body(x_ref, w_ref, b_ref, out_ref):
    out_ref[0] = (
        jnp.dot(x_ref[0], w_ref[...], preferred_element_type=jnp.float32)
        + b_ref[...]
    )


_PAR1 = pltpu.CompilerParams(dimension_semantics=("parallel",))
_PAR2 = pltpu.CompilerParams(dimension_semantics=("parallel", "parallel"))


def _proj(x, w3, b3, bn):
    return pl.pallas_call(
        _proj_body,
        compiler_params=_PAR2,
        grid=(_B, _N // bn),
        in_specs=[
            pl.BlockSpec((1, bn, _D), lambda b, n: (b, n, 0)),
            pl.BlockSpec((_D, 3 * _H), lambda b, n: (0, 0)),
            pl.BlockSpec((1, 3 * _H), lambda b, n: (0, 0)),
        ],
        out_specs=pl.BlockSpec((1, bn, 3 * _H), lambda b, n: (b, n, 0)),
        out_shape=jax.ShapeDtypeStruct((_B, _N, 3 * _H), jnp.float32),
    )(x, w3, b3)


def _slots_body(inv_t_ref, k_ref, v_ref, out_ref):
    kmat = k_ref[0]
    vmat = v_ref[0]
    inv_t = inv_t_ref[0, 0]
    r = jax.lax.broadcasted_iota(jnp.int32, (_S, _H), 0)
    c = jax.lax.broadcasted_iota(jnp.int32, (_S, _H), 1)
    q = jnp.where(r == c, 1.0, 0.0) + jnp.where(r - _H == c, -1.0, 0.0)
    for _ in range(_ITERS):
        dots = jax.lax.dot_general(
            q, kmat, (((1,), (1,)), ((), ())),
            preferred_element_type=jnp.float32) * _SCALE
        p = _topk_softmax(dots, _K1, inv_t)
        s = jnp.dot(p, vmat, preferred_element_type=jnp.float32)
        nrm = jnp.sqrt(jnp.sum(s * s, axis=1, keepdims=True))
        q = s / jnp.maximum(nrm, 1e-12)
    out_ref[0] = q


def _slots(kk, vv, inv_t1):
    return pl.pallas_call(
        _slots_body,
        compiler_params=_PAR1,
        grid=(_B,),
        in_specs=[
            pl.BlockSpec(memory_space=pltpu.SMEM),
            pl.BlockSpec((1, _N, _H), lambda b: (b, 0, 0)),
            pl.BlockSpec((1, _N, _H), lambda b: (b, 0, 0)),
        ],
        out_specs=pl.BlockSpec((1, _S, _H), lambda b: (b, 0, 0)),
        out_shape=jax.ShapeDtypeStruct((_B, _S, _H), jnp.float32),
    )(inv_t1, kk, vv)


def _stage2_body(inv_t_ref, q_ref, s_ref, wbv_ref, bbv_ref, x_ref,
                 g_ref, beta_ref, out_ref):
    slots = s_ref[0]
    inv_t = inv_t_ref[0, 0]
    slots_v = (jnp.dot(slots.astype(jnp.bfloat16), wbv_ref[...],
                       preferred_element_type=jnp.float32) + bbv_ref[...])
    logits = jax.lax.dot_general(
        q_ref[0], slots, (((1,), (1,)), ((), ())),
        preferred_element_type=jnp.float32) * _SCALE
    p = _topk_softmax(logits, _K2, inv_t)
    y = jnp.dot(p.astype(jnp.bfloat16), slots_v.astype(jnp.bfloat16),
                preferred_element_type=jnp.float32) + x_ref[0]
    mu = jnp.mean(y, axis=1, keepdims=True)
    var = jnp.mean((y - mu) ** 2, axis=1, keepdims=True)
    out_ref[0] = ((y - mu) / jnp.sqrt(var + _EPS_LN) * g_ref[...]
                  + beta_ref[...])


def _stage2(emb_q, slots, wbv, bbv, x, g1, beta1, inv_t2):
    return pl.pallas_call(
        _stage2_body,
        compiler_params=_PAR1,
        grid=(_B,),
        in_specs=[
            pl.BlockSpec(memory_space=pltpu.SMEM),
            pl.BlockSpec((1, _N, _H), lambda b: (b, 0, 0)),
            pl.BlockSpec((1, _S, _H), lambda b: (b, 0, 0)),
            pl.BlockSpec((_H, _D), lambda b: (0, 0)),
            pl.BlockSpec((1, _D), lambda b: (0, 0)),
            pl.BlockSpec((1, _N, _D), lambda b: (b, 0, 0)),
            pl.BlockSpec((1, _D), lambda b: (0, 0)),
            pl.BlockSpec((1, _D), lambda b: (0, 0)),
        ],
        out_specs=pl.BlockSpec((1, _N, _D), lambda b: (b, 0, 0)),
        out_shape=jax.ShapeDtypeStruct((_B, _N, _D), jnp.float32),
    )(inv_t2, emb_q, slots, wbv, bbv, x, g1, beta1)


def _ffn_body(x_ref, w1_ref, b1_ref, w2_ref, b2_ref, g_ref, beta_ref,
              out_ref):
    x = x_ref[0]
    h = jnp.dot(x.astype(jnp.bfloat16), w1_ref[...],
                preferred_element_type=jnp.float32) + b1_ref[...]
    h = 0.5 * h * (1.0 + jax.lax.erf(h * (2.0 ** -0.5)))
    y = x + jnp.dot(h.astype(jnp.bfloat16), w2_ref[...],
                    preferred_element_type=jnp.float32) + b2_ref[...]
    mu = jnp.mean(y, axis=1, keepdims=True)
    var = jnp.mean((y - mu) ** 2, axis=1, keepdims=True)
    out_ref[0] = ((y - mu) / jnp.sqrt(var + _EPS_LN) * g_ref[...]
                  + beta_ref[...])


def _ffn(x, w1, b1, w2, b2, g2, beta2, bn):
    return pl.pallas_call(
        _ffn_body,
        compiler_params=_PAR2,
        grid=(_B, _N // bn),
        in_specs=[
            pl.BlockSpec((1, bn, _D), lambda b, n: (b, n, 0)),
            pl.BlockSpec((_D, 2 * _D), lambda b, n: (0, 0)),
            pl.BlockSpec((1, 2 * _D), lambda b, n: (0, 0)),
            pl.BlockSpec((2 * _D, _D), lambda b, n: (0, 0)),
            pl.BlockSpec((1, _D), lambda b, n: (0, 0)),
            pl.BlockSpec((1, _D), lambda b, n: (0, 0)),
            pl.BlockSpec((1, _D), lambda b, n: (0, 0)),
        ],
        out_specs=pl.BlockSpec((1, bn, _D), lambda b, n: (b, n, 0)),
        out_shape=jax.ShapeDtypeStruct((_B, _N, _D), jnp.float32),
    )(x, w1, b1, w2, b2, g2, beta2)


def kernel(inputs, Wk, bk, Wv, bv, WQ, bQ, Wbv, bbv, g1, beta1, W1, b1,
           W2, b2, g2, beta2, t1, t2):
    w3 = jnp.concatenate([Wk, Wv, WQ], axis=1)
    b3 = jnp.concatenate([bk, bv, bQ])[None, :]
    kvq = _proj(inputs, w3, b3, bn=512)
    kk = kvq[:, :, :_H]
    vv = kvq[:, :, _H:2 * _H]
    emb_q = kvq[:, :, 2 * _H:]
    inv_t1 = jnp.reshape(1.0 / t1, (1, 1))
    inv_t2 = jnp.reshape(1.0 / t2, (1, 1))
    slots = _slots(kk, vv, inv_t1)
    emb = _stage2(emb_q, slots, Wbv.astype(jnp.bfloat16), bbv[None, :],
                  inputs, g1[None, :], beta1[None, :], inv_t2)
    out = _ffn(emb, W1.astype(jnp.bfloat16), b1[None, :],
               W2.astype(jnp.bfloat16), b2[None, :], g2[None, :],
               beta2[None, :], bn=512)
    return out


# speedup vs baseline: 1.6141x; 1.6131x over previous
import jax
import jax.numpy as jnp
from jax.experimental import pallas as pl
from jax.experimental.pallas import tpu as pltpu

_B, _N, _D, _H = 4, 2048, 1024, 256
_S = 2 * _H
_ITERS = 3
_K1 = 64
_K2 = _S // 3
_SCALE = _H ** -0.5
_MINT = -2147483648
_EPS_LN = 1e-5


_BISECT_STEPS = 16


def _topk_softmax(dots, k, inv_t):
    hi = jnp.max(dots, axis=1, keepdims=True)
    lo = jnp.min(dots, axis=1, keepdims=True)
    m = hi
    for _ in range(_BISECT_STEPS):
        mid = 0.5 * (lo + hi)
        cnt = jnp.sum((dots >= mid).astype(jnp.float32), axis=1,
                      keepdims=True)
        pick = cnt >= k
        lo = jnp.where(pick, mid, lo)
        hi = jnp.where(pick, hi, mid)
    mask = dots >= lo
    e = jnp.where(mask, jnp.exp((dots - m) * inv_t), 0.0)
    return e / jnp.sum(e, axis=1, keepdims=True)


def _proj_body(x_ref, w_ref, b_ref, out_ref):
    out_ref[0] = (
        jnp.dot(x_ref[0], w_ref[...], preferred_element_type=jnp.float32)
        + b_ref[...]
    )


_PAR1 = pltpu.CompilerParams(dimension_semantics=("parallel",))
_PAR2 = pltpu.CompilerParams(dimension_semantics=("parallel", "parallel"))


def _proj(x, w3, b3, bn):
    return pl.pallas_call(
        _proj_body,
        compiler_params=_PAR2,
        grid=(_B, _N // bn),
        in_specs=[
            pl.BlockSpec((1, bn, _D), lambda b, n: (b, n, 0)),
            pl.BlockSpec((_D, 3 * _H), lambda b, n: (0, 0)),
            pl.BlockSpec((1, 3 * _H), lambda b, n: (0, 0)),
        ],
        out_specs=pl.BlockSpec((1, bn, 3 * _H), lambda b, n: (b, n, 0)),
        out_shape=jax.ShapeDtypeStruct((_B, _N, 3 * _H), jnp.float32),
    )(x, w3, b3)


def _slots_body(inv_t_ref, k_ref, v_ref, out_ref):
    kmat = k_ref[0]
    vmat = v_ref[0]
    inv_t = inv_t_ref[0, 0]
    r = jax.lax.broadcasted_iota(jnp.int32, (_S, _H), 0)
    c = jax.lax.broadcasted_iota(jnp.int32, (_S, _H), 1)
    q = jnp.where(r == c, 1.0, 0.0) + jnp.where(r - _H == c, -1.0, 0.0)
    for _ in range(_ITERS):
        dots = jax.lax.dot_general(
            q, kmat, (((1,), (1,)), ((), ())),
            preferred_element_type=jnp.float32) * _SCALE
        p = _topk_softmax(dots, _K1, inv_t)
        s = jnp.dot(p, vmat, preferred_element_type=jnp.float32)
        nrm = jnp.sqrt(jnp.sum(s * s, axis=1, keepdims=True))
        q = s / jnp.maximum(nrm, 1e-12)
    out_ref[0] = q


def _slots(kk, vv, inv_t1):
    return pl.pallas_call(
        _slots_body,
        compiler_params=_PAR1,
        grid=(_B,),
        in_specs=[
            pl.BlockSpec(memory_space=pltpu.SMEM),
            pl.BlockSpec((1, _N, _H), lambda b: (b, 0, 0)),
            pl.BlockSpec((1, _N, _H), lambda b: (b, 0, 0)),
        ],
        out_specs=pl.BlockSpec((1, _S, _H), lambda b: (b, 0, 0)),
        out_shape=jax.ShapeDtypeStruct((_B, _S, _H), jnp.float32),
    )(inv_t1, kk, vv)


def _stage2_body(inv_t_ref, q_ref, s_ref, wbv_ref, bbv_ref, x_ref,
                 g_ref, beta_ref, out_ref):
    slots = s_ref[0]
    inv_t = inv_t_ref[0, 0]
    slots_v = (jnp.dot(slots.astype(jnp.bfloat16), wbv_ref[...],
                       preferred_element_type=jnp.float32) + bbv_ref[...])
    logits = jax.lax.dot_general(
        q_ref[0], slots, (((1,), (1,)), ((), ())),
        preferred_element_type=jnp.float32) * _SCALE
    p = _topk_softmax(logits, _K2, inv_t)
    y = jnp.dot(p.astype(jnp.bfloat16), slots_v.astype(jnp.bfloat16),
                preferred_element_type=jnp.float32) + x_ref[0]
    mu = jnp.mean(y, axis=1, keepdims=True)
    var = jnp.mean((y - mu) ** 2, axis=1, keepdims=True)
    out_ref[0] = ((y - mu) / jnp.sqrt(var + _EPS_LN) * g_ref[...]
                  + beta_ref[...])


def _stage2(emb_q, slots, wbv, bbv, x, g1, beta1, inv_t2):
    return pl.pallas_call(
        _stage2_body,
        compiler_params=_PAR1,
        grid=(_B,),
        in_specs=[
            pl.BlockSpec(memory_space=pltpu.SMEM),
            pl.BlockSpec((1, _N, _H), lambda b: (b, 0, 0)),
            pl.BlockSpec((1, _S, _H), lambda b: (b, 0, 0)),
            pl.BlockSpec((_H, _D), lambda b: (0, 0)),
            pl.BlockSpec((1, _D), lambda b: (0, 0)),
            pl.BlockSpec((1, _N, _D), lambda b: (b, 0, 0)),
            pl.BlockSpec((1, _D), lambda b: (0, 0)),
            pl.BlockSpec((1, _D), lambda b: (0, 0)),
        ],
        out_specs=pl.BlockSpec((1, _N, _D), lambda b: (b, 0, 0)),
        out_shape=jax.ShapeDtypeStruct((_B, _N, _D), jnp.float32),
    )(inv_t2, emb_q, slots, wbv, bbv, x, g1, beta1)


def _ffn_body(x_ref, w1_ref, b1_ref, w2_ref, b2_ref, g_ref, beta_ref,
              out_ref):
    x = x_ref[0]
    h = jnp.dot(x.astype(jnp.bfloat16), w1_ref[...],
                preferred_element_type=jnp.float32) + b1_ref[...]
    h = 0.5 * h * (1.0 + jax.lax.erf(h * (2.0 ** -0.5)))
    y = x + jnp.dot(h.astype(jnp.bfloat16), w2_ref[...],
                    preferred_element_type=jnp.float32) + b2_ref[...]
    mu = jnp.mean(y, axis=1, keepdims=True)
    var = jnp.mean((y - mu) ** 2, axis=1, keepdims=True)
    out_ref[0] = ((y - mu) / jnp.sqrt(var + _EPS_LN) * g_ref[...]
                  + beta_ref[...])


def _ffn(x, w1, b1, w2, b2, g2, beta2, bn):
    return pl.pallas_call(
        _ffn_body,
        compiler_params=_PAR2,
        grid=(_B, _N // bn),
        in_specs=[
            pl.BlockSpec((1, bn, _D), lambda b, n: (b, n, 0)),
            pl.BlockSpec((_D, 2 * _D), lambda b, n: (0, 0)),
            pl.BlockSpec((1, 2 * _D), lambda b, n: (0, 0)),
            pl.BlockSpec((2 * _D, _D), lambda b, n: (0, 0)),
            pl.BlockSpec((1, _D), lambda b, n: (0, 0)),
            pl.BlockSpec((1, _D), lambda b, n: (0, 0)),
            pl.BlockSpec((1, _D), lambda b, n: (0, 0)),
        ],
        out_specs=pl.BlockSpec((1, bn, _D), lambda b, n: (b, n, 0)),
        out_shape=jax.ShapeDtypeStruct((_B, _N, _D), jnp.float32),
    )(x, w1, b1, w2, b2, g2, beta2)


def kernel(inputs, Wk, bk, Wv, bv, WQ, bQ, Wbv, bbv, g1, beta1, W1, b1,
           W2, b2, g2, beta2, t1, t2):
    w3 = jnp.concatenate([Wk, Wv, WQ], axis=1)
    b3 = jnp.concatenate([bk, bv, bQ])[None, :]
    kvq = _proj(inputs, w3, b3, bn=512)
    kk = kvq[:, :, :_H]
    vv = kvq[:, :, _H:2 * _H]
    emb_q = kvq[:, :, 2 * _H:]
    inv_t1 = jnp.reshape(1.0 / t1, (1, 1))
    inv_t2 = jnp.reshape(1.0 / t2, (1, 1))
    slots = _slots(kk, vv, inv_t1)
    emb = _stage2(emb_q, slots, Wbv.astype(jnp.bfloat16), bbv[None, :],
                  inputs, g1[None, :], beta1[None, :], inv_t2)
    out = _ffn(emb, W1.astype(jnp.bfloat16), b1[None, :],
               W2.astype(jnp.bfloat16), b2[None, :], g2[None, :],
               beta2[None, :], bn=512)
    return out


# speedup vs baseline: 1.7521x; 1.0855x over previous
import jax
import jax.numpy as jnp
from jax.experimental import pallas as pl
from jax.experimental.pallas import tpu as pltpu

_B, _N, _D, _H = 4, 2048, 1024, 256
_S = 2 * _H
_ITERS = 3
_K1 = 64
_K2 = _S // 3
_SCALE = _H ** -0.5
_MINT = -2147483648
_EPS_LN = 1e-5


_BISECT_STEPS = 16


def _topk_softmax(dots, k, inv_t):
    hi = jnp.max(dots, axis=1, keepdims=True)
    lo = jnp.min(dots, axis=1, keepdims=True)
    m = hi
    for _ in range(_BISECT_STEPS):
        mid = 0.5 * (lo + hi)
        cnt = jnp.sum((dots >= mid).astype(jnp.float32), axis=1,
                      keepdims=True)
        pick = cnt >= k
        lo = jnp.where(pick, mid, lo)
        hi = jnp.where(pick, hi, mid)
    mask = dots >= lo
    e = jnp.where(mask, jnp.exp((dots - m) * inv_t), 0.0)
    return e / jnp.sum(e, axis=1, keepdims=True)


def _proj_body(x_ref, w_ref, b_ref, out_ref):
    out_ref[0] = (
        jnp.dot(x_ref[0], w_ref[...], preferred_element_type=jnp.float32)
        + b_ref[...]
    )


_PAR1 = pltpu.CompilerParams(dimension_semantics=("parallel",))
_PAR2 = pltpu.CompilerParams(dimension_semantics=("parallel", "parallel"))


def _proj(x, w3, b3, bn):
    return pl.pallas_call(
        _proj_body,
        compiler_params=_PAR2,
        grid=(_B, _N // bn),
        in_specs=[
            pl.BlockSpec((1, bn, _D), lambda b, n: (b, n, 0)),
            pl.BlockSpec((_D, 3 * _H), lambda b, n: (0, 0)),
            pl.BlockSpec((1, 3 * _H), lambda b, n: (0, 0)),
        ],
        out_specs=pl.BlockSpec((1, bn, 3 * _H), lambda b, n: (b, n, 0)),
        out_shape=jax.ShapeDtypeStruct((_B, _N, 3 * _H), jnp.float32),
    )(x, w3, b3)


def _slots_body(inv_t_ref, k_ref, v_ref, out_ref):
    kmat = k_ref[0]
    vmat = v_ref[0]
    inv_t = inv_t_ref[0, 0]
    r = jax.lax.broadcasted_iota(jnp.int32, (_S, _H), 0)
    c = jax.lax.broadcasted_iota(jnp.int32, (_S, _H), 1)
    q = jnp.where(r == c, 1.0, 0.0) + jnp.where(r - _H == c, -1.0, 0.0)
    for _ in range(_ITERS):
        dots = jax.lax.dot_general(
            q, kmat, (((1,), (1,)), ((), ())),
            preferred_element_type=jnp.float32) * _SCALE
        p = _topk_softmax(dots, _K1, inv_t)
        s = jnp.dot(p, vmat, preferred_element_type=jnp.float32)
        nrm = jnp.sqrt(jnp.sum(s * s, axis=1, keepdims=True))
        q = s / jnp.maximum(nrm, 1e-12)
    out_ref[0] = q


def _slots(kk, vv, inv_t1):
    return pl.pallas_call(
        _slots_body,
        compiler_params=_PAR1,
        grid=(_B,),
        in_specs=[
            pl.BlockSpec(memory_space=pltpu.SMEM),
            pl.BlockSpec((1, _N, _H), lambda b: (b, 0, 0)),
            pl.BlockSpec((1, _N, _H), lambda b: (b, 0, 0)),
        ],
        out_specs=pl.BlockSpec((1, _S, _H), lambda b: (b, 0, 0)),
        out_shape=jax.ShapeDtypeStruct((_B, _S, _H), jnp.float32),
    )(inv_t1, kk, vv)


def _stage2_body(inv_t_ref, q_ref, s_ref, wbv_ref, bbv_ref, x_ref,
                 g_ref, beta_ref, out_ref):
    slots = s_ref[0]
    inv_t = inv_t_ref[0, 0]
    slots_v = (jnp.dot(slots, wbv_ref[...],
                       preferred_element_type=jnp.float32) + bbv_ref[...])
    logits = jax.lax.dot_general(
        q_ref[0], slots, (((1,), (1,)), ((), ())),
        preferred_element_type=jnp.float32) * _SCALE
    p = _topk_softmax(logits, _K2, inv_t)
    y = jnp.dot(p, slots_v, preferred_element_type=jnp.float32) + x_ref[0]
    mu = jnp.mean(y, axis=1, keepdims=True)
    var = jnp.mean((y - mu) ** 2, axis=1, keepdims=True)
    out_ref[0] = ((y - mu) / jnp.sqrt(var + _EPS_LN) * g_ref[...]
                  + beta_ref[...])


def _stage2(emb_q, slots, wbv, bbv, x, g1, beta1, inv_t2):
    return pl.pallas_call(
        _stage2_body,
        compiler_params=_PAR1,
        grid=(_B,),
        in_specs=[
            pl.BlockSpec(memory_space=pltpu.SMEM),
            pl.BlockSpec((1, _N, _H), lambda b: (b, 0, 0)),
            pl.BlockSpec((1, _S, _H), lambda b: (b, 0, 0)),
            pl.BlockSpec((_H, _D), lambda b: (0, 0)),
            pl.BlockSpec((1, _D), lambda b: (0, 0)),
            pl.BlockSpec((1, _N, _D), lambda b: (b, 0, 0)),
            pl.BlockSpec((1, _D), lambda b: (0, 0)),
            pl.BlockSpec((1, _D), lambda b: (0, 0)),
        ],
        out_specs=pl.BlockSpec((1, _N, _D), lambda b: (b, 0, 0)),
        out_shape=jax.ShapeDtypeStruct((_B, _N, _D), jnp.float32),
    )(inv_t2, emb_q, slots, wbv, bbv, x, g1, beta1)


def _ffn_body(x_ref, w1_ref, b1_ref, w2_ref, b2_ref, g_ref, beta_ref,
              out_ref):
    x = x_ref[0]
    h = jnp.dot(x, w1_ref[...],
                preferred_element_type=jnp.float32) + b1_ref[...]
    h = 0.5 * h * (1.0 + jax.lax.erf(h * (2.0 ** -0.5)))
    y = x + jnp.dot(h, w2_ref[...],
                    preferred_element_type=jnp.float32) + b2_ref[...]
    mu = jnp.mean(y, axis=1, keepdims=True)
    var = jnp.mean((y - mu) ** 2, axis=1, keepdims=True)
    out_ref[0] = ((y - mu) / jnp.sqrt(var + _EPS_LN) * g_ref[...]
                  + beta_ref[...])


def _ffn(x, w1, b1, w2, b2, g2, beta2, bn):
    return pl.pallas_call(
        _ffn_body,
        compiler_params=_PAR2,
        grid=(_B, _N // bn),
        in_specs=[
            pl.BlockSpec((1, bn, _D), lambda b, n: (b, n, 0)),
            pl.BlockSpec((_D, 2 * _D), lambda b, n: (0, 0)),
            pl.BlockSpec((1, 2 * _D), lambda b, n: (0, 0)),
            pl.BlockSpec((2 * _D, _D), lambda b, n: (0, 0)),
            pl.BlockSpec((1, _D), lambda b, n: (0, 0)),
            pl.BlockSpec((1, _D), lambda b, n: (0, 0)),
            pl.BlockSpec((1, _D), lambda b, n: (0, 0)),
        ],
        out_specs=pl.BlockSpec((1, bn, _D), lambda b, n: (b, n, 0)),
        out_shape=jax.ShapeDtypeStruct((_B, _N, _D), jnp.float32),
    )(x, w1, b1, w2, b2, g2, beta2)


def kernel(inputs, Wk, bk, Wv, bv, WQ, bQ, Wbv, bbv, g1, beta1, W1, b1,
           W2, b2, g2, beta2, t1, t2):
    w3 = jnp.concatenate([Wk, Wv, WQ], axis=1)
    b3 = jnp.concatenate([bk, bv, bQ])[None, :]
    kvq = _proj(inputs, w3, b3, bn=512)
    kk = kvq[:, :, :_H]
    vv = kvq[:, :, _H:2 * _H]
    emb_q = kvq[:, :, 2 * _H:]
    inv_t1 = jnp.reshape(1.0 / t1, (1, 1))
    inv_t2 = jnp.reshape(1.0 / t2, (1, 1))
    slots = _slots(kk, vv, inv_t1)
    emb = _stage2(emb_q, slots, Wbv, bbv[None, :],
                  inputs, g1[None, :], beta1[None, :], inv_t2)
    out = _ffn(emb, W1, b1[None, :], W2, b2[None, :], g2[None, :],
               beta2[None, :], bn=512)
    return out
